# baseline (speedup 1.0000x reference)
import functools

import jax
import jax.numpy as jnp
from jax import lax
from jax.experimental import pallas as pl
from jax.experimental.pallas import tpu as pltpu
from jax.experimental.pallas import tpu_sc as plsc

_VOCABS = [9, 16, 7, 15, 6, 5, 2, 40]
_DIMS = [3, 5, 2, 5, 2, 2, 2, 13]
_OFFS = [0, 3, 8, 10, 15, 17, 19, 21]
_VBASE = [0, 9, 25, 32, 47, 53, 58, 60]
_B = 4096
_H = 50
_GROUPS = [(1, 3), (7, 6), (0, 2), (4, 5)]
_GROUP_BASE = [0, 240, 320, 383]
_CROWS = 416

_NW = 32
_BPW = _B // _NW
_NTROWS = 58
_NTW = _NTROWS * _BPW


def _tc_build_body(e_ref, wall_ref, x_ref, c_ref, nt_ref):
    wall = wall_ref[...]
    w1 = wall[0:40, :]
    e = e_ref[...]
    fused = [
        jnp.dot(e[_VBASE[i]:_VBASE[i] + _VOCABS[i], 0:_DIMS[i]],
                w1[_OFFS[i]:_OFFS[i] + _DIMS[i], :],
                preferred_element_type=jnp.float32)
        for i in range(8)
    ]
    f = jnp.concatenate(fused, axis=0)
    r = lax.broadcasted_iota(jnp.int32, (_CROWS, 100), 0).astype(jnp.float32)
    col = lax.broadcasted_iota(jnp.int32, (_CROWS, 100), 1).astype(jnp.float32)
    s = jnp.zeros((_CROWS, 100), jnp.float32)
    for gi, (ta, tb) in enumerate(_GROUPS):
        va, vb = _VOCABS[ta], _VOCABS[tb]
        rr = r - _GROUP_BASE[gi]
        a = jnp.floor((rr + 0.5) * (1.0 / vb))
        bdig = rr - a * vb
        inr = (rr >= 0.0) & (rr < float(va * vb))
        hit = ((col == a + _VBASE[ta]) | (col == bdig + _VBASE[tb])) & inr
        s = s + hit.astype(jnp.float32)
    c = jnp.dot(s, f, preferred_element_type=jnp.float32)
    rr1 = lax.broadcasted_iota(jnp.int32, (_CROWS, _H), 0)
    c = jnp.where(rr1 == 413, wall[41:42, :], c)
    c = jnp.where(rr1 == 414, wall[42:43, :], c)
    c_ref[...] = c
    xnum1 = jnp.concatenate(
        [x_ref[...][:, 8:14], jnp.ones((_B, 1), jnp.float32)], axis=1)
    n = lax.dot_general(wall[34:41, :], xnum1, (((0,), (1,)), ((), ())),
                        preferred_element_type=jnp.float32)
    i8 = lax.broadcasted_iota(jnp.int32, (8, 14), 0)
    i14 = lax.broadcasted_iota(jnp.int32, (8, 14), 1)
    sel8 = (i8 == i14).astype(jnp.float32)
    xc = lax.dot_general(sel8, x_ref[...], (((1,), (1,)), ((), ())),
                         preferred_element_type=jnp.float32)
    for w in range(_NW):
        nt_ref[w, 0:_H, :] = n[:, w * _BPW:(w + 1) * _BPW]
        nt_ref[w, _H:_NTROWS, :] = xc[:, w * _BPW:(w + 1) * _BPW]


_SC_MESH = plsc.VectorSubcoreMesh(core_axis_name="c", subcore_axis_name="s",
                                  num_cores=2, num_subcores=16)


@functools.partial(
    pl.kernel,
    out_type=jax.ShapeDtypeStruct((_B,), jnp.float32),
    mesh=_SC_MESH,
    compiler_params=pltpu.CompilerParams(needs_layout_passes=False),
    scratch_types=[
        pltpu.VMEM((_CROWS * _H,), jnp.float32),
        pltpu.VMEM((_NTW,), jnp.float32),
        pltpu.VMEM((_BPW,), jnp.float32),
    ],
)
def _sc_forward(c_hbm, nt_hbm, out_hbm, c_v, n_v, o_v):
    wid = lax.axis_index("s") * 2 + lax.axis_index("c")
    base = wid * _BPW
    pltpu.sync_copy(c_hbm, c_v)
    pltpu.sync_copy(nt_hbm.at[pl.ds(wid * _NTW, _NTW)], n_v)

    fb = []
    for g in range(8):
        ci = [n_v[pl.ds((_H + t) * _BPW + g * 16, 16)].astype(jnp.int32)
              for t in range(8)]
        fb.append([(ci[ta] * _VOCABS[tb] + ci[tb] + _GROUP_BASE[gi]) * _H
                   for gi, (ta, tb) in enumerate(_GROUPS)])
    out_acc = [jnp.zeros((16,), jnp.float32) for _ in range(8)]
    for j in range(_H):
        w2j = plsc.load_gather(
            c_v, [jnp.full((16,), 413 * _H + j, jnp.int32)])
        for g in range(8):
            acc = plsc.load_gather(c_v, [fb[g][0] + j])
            acc = acc + plsc.load_gather(c_v, [fb[g][1] + j])
            acc = acc + plsc.load_gather(c_v, [fb[g][2] + j])
            acc = acc + plsc.load_gather(c_v, [fb[g][3] + j])
            acc = acc + n_v[pl.ds(j * _BPW + g * 16, 16)]
            acc = jnp.maximum(acc, 0.0)
            out_acc[g] = out_acc[g] + acc * w2j
    b2 = plsc.load_gather(c_v, [jnp.full((16,), 414 * _H, jnp.int32)])
    for g in range(8):
        z = out_acc[g] + b2
        o_v[pl.ds(g * 16, 16)] = 1.0 / (1.0 + jnp.exp(-z))
    pltpu.sync_copy(o_v, out_hbm.at[pl.ds(base, _BPW)])


def kernel(x, emb0, emb1, emb2, emb3, emb4, emb5, emb6, emb7, W1, b1, W2, b2):
    embs = [emb0, emb1, emb2, emb3, emb4, emb5, emb6, emb7]
    e = jnp.concatenate(
        [jnp.pad(embs[i], ((0, 0), (0, 16 - _DIMS[i]))) for i in range(8)],
        axis=0)
    wall = jnp.concatenate(
        [W1, b1.reshape(1, _H), W2.reshape(1, _H),
         jnp.pad(b2.reshape(1, 1), ((0, 0), (0, _H - 1)))], axis=0)
    c, nt = pl.pallas_call(
        _tc_build_body,
        out_shape=(jax.ShapeDtypeStruct((_CROWS, _H), jnp.float32),
                   jax.ShapeDtypeStruct((_NW, _NTROWS, _BPW), jnp.float32)),
    )(e, wall, x)
    out = _sc_forward(c.reshape(_CROWS * _H), nt.reshape(_NW * _NTW))
    return out.reshape(_B, 1)


# speedup vs baseline: 1.2022x; 1.2022x over previous
import functools

import jax
import jax.numpy as jnp
from jax import lax
from jax.experimental import pallas as pl
from jax.experimental.pallas import tpu as pltpu
from jax.experimental.pallas import tpu_sc as plsc

_VOCABS = [9, 16, 7, 15, 6, 5, 2, 40]
_DIMS = [3, 5, 2, 5, 2, 2, 2, 13]
_OFFS = [0, 3, 8, 10, 15, 17, 19, 21]
_VBASE = [0, 9, 25, 32, 47, 53, 58, 60]
_B = 4096
_H = 50
_GROUPS = [(1, 3), (7, 6), (0, 2), (4, 5)]
_GROUP_BASE = [0, 240, 320, 383]
_R_W2, _R_B2, _R_B1, _R_WN = 413, 414, 415, 416
_CROWS = 424

_NW = 32
_BPW = _B // _NW


def _tc_build_body(e_ref, wall_ref, c_ref):
    wall = wall_ref[...]
    w1 = wall[0:40, :]
    e = e_ref[...]
    fused = [
        jnp.dot(e[_VBASE[i]:_VBASE[i] + _VOCABS[i], 0:_DIMS[i]],
                w1[_OFFS[i]:_OFFS[i] + _DIMS[i], :],
                preferred_element_type=jnp.float32)
        for i in range(8)
    ]
    f = jnp.concatenate(fused, axis=0)
    r = lax.broadcasted_iota(jnp.int32, (_CROWS, 100), 0).astype(jnp.float32)
    col = lax.broadcasted_iota(jnp.int32, (_CROWS, 100), 1).astype(jnp.float32)
    s = jnp.zeros((_CROWS, 100), jnp.float32)
    for gi, (ta, tb) in enumerate(_GROUPS):
        va, vb = _VOCABS[ta], _VOCABS[tb]
        rr = r - _GROUP_BASE[gi]
        a = jnp.floor((rr + 0.5) * (1.0 / vb))
        bdig = rr - a * vb
        inr = (rr >= 0.0) & (rr < float(va * vb))
        hit = ((col == a + _VBASE[ta]) | (col == bdig + _VBASE[tb])) & inr
        s = s + hit.astype(jnp.float32)
    c = jnp.dot(s, f, preferred_element_type=jnp.float32)
    rr1 = lax.broadcasted_iota(jnp.int32, (_CROWS, _H), 0)
    c = jnp.where(rr1 == _R_W2, wall[41:42, :], c)
    c = jnp.where(rr1 == _R_B2, wall[42:43, :], c)
    c = jnp.where(rr1 == _R_B1, wall[40:41, :], c)
    for k in range(6):
        c = jnp.where(rr1 == _R_WN + k, wall[34 + k:35 + k, :], c)
    c_ref[...] = c


_SC_MESH = plsc.VectorSubcoreMesh(core_axis_name="c", subcore_axis_name="s",
                                  num_cores=2, num_subcores=16)


def _splat(c_v, addr):
    return plsc.load_gather(c_v, [jnp.full((16,), addr, jnp.int32)])


@functools.partial(
    pl.kernel,
    out_type=jax.ShapeDtypeStruct((_B,), jnp.float32),
    mesh=_SC_MESH,
    compiler_params=pltpu.CompilerParams(needs_layout_passes=False),
    scratch_types=[
        pltpu.VMEM((_CROWS * _H,), jnp.float32),
        pltpu.VMEM((_BPW * 14,), jnp.float32),
        pltpu.VMEM((_BPW,), jnp.float32),
    ],
)
def _sc_forward(c_hbm, x_hbm, out_hbm, c_v, x_v, o_v):
    wid = lax.axis_index("s") * 2 + lax.axis_index("c")
    base = wid * _BPW
    pltpu.sync_copy(c_hbm, c_v)
    pltpu.sync_copy(x_hbm.at[pl.ds(base * 14, _BPW * 14)], x_v)

    lane = lax.iota(jnp.int32, 16)

    def step(it, carry):
        gs = [2 * it, 2 * it + 1]
        fb, xnum, out_acc = [], [], []
        for g in gs:
            rowbase = (lane + g * 16) * 14
            ci = [plsc.load_gather(x_v, [rowbase + t]).astype(jnp.int32)
                  for t in range(8)]
            fb.append([(ci[ta] * _VOCABS[tb] + ci[tb] + _GROUP_BASE[gi]) * _H
                       for gi, (ta, tb) in enumerate(_GROUPS)])
            xnum.append([plsc.load_gather(x_v, [rowbase + 8 + k])
                         for k in range(6)])
            out_acc.append(jnp.zeros((16,), jnp.float32))
        for j in range(_H):
            w2j = _splat(c_v, _R_W2 * _H + j)
            b1j = _splat(c_v, _R_B1 * _H + j)
            wnj = [_splat(c_v, (_R_WN + k) * _H + j) for k in range(6)]
            for i in range(2):
                acc = b1j + plsc.load_gather(c_v, [fb[i][0] + j])
                acc = acc + plsc.load_gather(c_v, [fb[i][1] + j])
                acc = acc + plsc.load_gather(c_v, [fb[i][2] + j])
                acc = acc + plsc.load_gather(c_v, [fb[i][3] + j])
                for k in range(6):
                    acc = acc + xnum[i][k] * wnj[k]
                acc = jnp.maximum(acc, 0.0)
                out_acc[i] = out_acc[i] + acc * w2j
        b2 = _splat(c_v, _R_B2 * _H)
        for i in range(2):
            z = out_acc[i] + b2
            o_v[pl.ds(gs[i] * 16, 16)] = 1.0 / (1.0 + jnp.exp(-z))
        return carry

    lax.fori_loop(0, 4, step, None)
    pltpu.sync_copy(o_v, out_hbm.at[pl.ds(base, _BPW)])


def kernel(x, emb0, emb1, emb2, emb3, emb4, emb5, emb6, emb7, W1, b1, W2, b2):
    embs = [emb0, emb1, emb2, emb3, emb4, emb5, emb6, emb7]
    e = jnp.zeros((100, 16), jnp.float32)
    for i in range(8):
        e = e.at[_VBASE[i]:_VBASE[i] + _VOCABS[i], 0:_DIMS[i]].set(embs[i])
    wall = jnp.concatenate(
        [W1, b1.reshape(1, _H), W2.reshape(1, _H),
         jnp.pad(b2.reshape(1, 1), ((0, 0), (0, _H - 1)))], axis=0)
    c = pl.pallas_call(
        _tc_build_body,
        out_shape=jax.ShapeDtypeStruct((_CROWS, _H), jnp.float32),
    )(e, wall)
    out = _sc_forward(c.reshape(_CROWS * _H), x.reshape(_B * 14))
    return out.reshape(_B, 1)
